# Initial kernel scaffold; baseline (speedup 1.0000x reference)
import jax
import jax.numpy as jnp
from jax.experimental import pallas as pl


def kernel(instance_feats, phrase_feats, connect_mats, phrase_clustered_indexs, p_s_W, p_s_b, s_p_W, s_p_b, p_o_W, p_o_b, o_p_W, o_p_b, inst_w1_W, inst_w1_b, inst_w2_W, inst_w2_b, phra_w1_W, phra_w1_b, phra_w2_W, phra_w2_b):
    raise NotImplementedError("write your pallas kernel here")


# speedup vs baseline: 1.1696x; 1.1696x over previous
import functools
import jax
import jax.numpy as jnp
from jax import lax
from jax.experimental import pallas as pl
from jax.experimental.pallas import tpu as pltpu
from jax.experimental.pallas import tpu_sc as plsc

F = 512
P = 32768
NI = 4096
NP = 16384

NC, NS, L = 2, 16, 16
NW = NC * NS
PER_W = P // NW
CH = 32
NCH = PER_W // CH


def _stage_body(phra, inst, clu_i, c0_i, c1_i, stp, sti0, sti1,
                idxbuf, rowbuf, sem):
    c = lax.axis_index("c")
    s = lax.axis_index("s")
    wid = s * NC + c
    base = wid * PER_W

    def do_array(table, idx_hbm, out_hbm):
        pltpu.sync_copy(idx_hbm.at[pl.ds(wid * NCH, NCH), :], idxbuf)

        def chunk(t, carry):
            pltpu.async_copy(table.at[idxbuf.at[t]], rowbuf, sem).wait()
            pltpu.sync_copy(rowbuf, out_hbm.at[pl.ds(base + t * CH, CH), :])
            return carry

        lax.fori_loop(0, NCH, chunk, 0)

    do_array(phra, clu_i, stp)
    do_array(inst, c0_i, sti0)
    do_array(inst, c1_i, sti1)


def _stage(phra, inst, clu, conn0, conn1):
    clu_i = clu.reshape(NW * NCH, CH)
    c0_i = conn0.reshape(NW * NCH, CH)
    c1_i = conn1.reshape(NW * NCH, CH)
    mesh = plsc.VectorSubcoreMesh(core_axis_name="c", subcore_axis_name="s")
    out = jax.ShapeDtypeStruct((P, F), jnp.float32)
    fn = pl.kernel(
        _stage_body,
        out_type=(out, out, out),
        mesh=mesh,
        scratch_types=[
            pltpu.VMEM((NCH, CH), jnp.int32),
            pltpu.VMEM((CH, F), jnp.float32),
            pltpu.SemaphoreType.DMA,
        ],
    )
    return fn(phra, inst, clu_i, c0_i, c1_i)


def _gates_body(stp_ref, sti0_ref, sti1_ref, wp_ref, wi0_ref, wi1_ref,
                b_ref, keep_ref, m1_ref, m2_ref, m3_ref, m4_ref):
    stp = stp_ref[...]
    sti0 = sti0_ref[...]
    sti1 = sti1_ref[...]
    mp = jnp.dot(stp.astype(jnp.bfloat16), wp_ref[...],
                 preferred_element_type=jnp.float32)
    mi0 = jnp.dot(sti0.astype(jnp.bfloat16), wi0_ref[...],
                  preferred_element_type=jnp.float32)
    mi1 = jnp.dot(sti1.astype(jnp.bfloat16), wi1_ref[...],
                  preferred_element_type=jnp.float32)
    b = b_ref[...]

    def gate(pre):
        return jnp.mean(jax.nn.sigmoid(jnp.maximum(pre, 0.0)), axis=1,
                        keepdims=True)

    g1 = gate(mp[:, 0 * F:1 * F] + mi0[:, 0:F] + b[0:1, :])
    g2 = gate(mp[:, 1 * F:2 * F] + mi1[:, 0:F] + b[1:2, :])
    g3 = gate(mp[:, 2 * F:3 * F] + mi1[:, F:2 * F] + b[2:3, :])
    g4 = gate(mp[:, 3 * F:4 * F] + mi0[:, F:2 * F] + b[3:4, :])
    keep = keep_ref[...]
    m1_ref[...] = stp * (g1 * keep)
    m2_ref[...] = stp * (g2 * keep)
    m3_ref[...] = sti1 * g3
    m4_ref[...] = sti0 * g4


def _gates(stp, sti0, sti1, wp, wi0, wi1, bcat, keepf, blk=512):
    out = jax.ShapeDtypeStruct((P, F), jnp.float32)
    return pl.pallas_call(
        _gates_body,
        grid=(P // blk,),
        in_specs=[
            pl.BlockSpec((blk, F), lambda i: (i, 0)),
            pl.BlockSpec((blk, F), lambda i: (i, 0)),
            pl.BlockSpec((blk, F), lambda i: (i, 0)),
            pl.BlockSpec((F, 4 * F), lambda i: (0, 0)),
            pl.BlockSpec((F, 2 * F), lambda i: (0, 0)),
            pl.BlockSpec((F, 2 * F), lambda i: (0, 0)),
            pl.BlockSpec((4, F), lambda i: (0, 0)),
            pl.BlockSpec((blk, 1), lambda i: (i, 0)),
        ],
        out_specs=[pl.BlockSpec((blk, F), lambda i: (i, 0))] * 4,
        out_shape=(out, out, out, out),
    )(stp, sti0, sti1, wp, wi0, wi1, bcat, keepf)


def _combine_body(accs_ref, cnts_ref, acco_ref, cnto_ref, x_ref, w1_ref, b1_ref,
                  w2_ref, b2_ref, o_ref):
    cs = cnts_ref[...]
    co = cnto_ref[...]
    ps = jnp.where(cs > 0, accs_ref[...] / jnp.where(cs > 0, cs, 1.0), 0.0)
    po = jnp.where(co > 0, acco_ref[...] / jnp.where(co > 0, co, 1.0), 0.0)
    summ = 0.5 * (ps + po)
    h1 = jnp.maximum(
        jnp.dot(summ, w1_ref[...], preferred_element_type=jnp.float32)
        + b1_ref[...], 0.0)
    h2 = jnp.maximum(
        jnp.dot(x_ref[...], w2_ref[...], preferred_element_type=jnp.float32)
        + b2_ref[...], 0.0)
    o_ref[...] = summ + h1 + h2


def _combine(acc_s, cnt_s, acc_o, cnt_o, x, w1, b1, w2, b2, blk=512):
    n = x.shape[0]
    return pl.pallas_call(
        _combine_body,
        grid=(n // blk,),
        in_specs=[
            pl.BlockSpec((blk, F), lambda i: (i, 0)),
            pl.BlockSpec((blk, 1), lambda i: (i, 0)),
            pl.BlockSpec((blk, F), lambda i: (i, 0)),
            pl.BlockSpec((blk, 1), lambda i: (i, 0)),
            pl.BlockSpec((blk, F), lambda i: (i, 0)),
            pl.BlockSpec((F, F), lambda i: (0, 0)),
            pl.BlockSpec((1, F), lambda i: (0, 0)),
            pl.BlockSpec((F, F), lambda i: (0, 0)),
            pl.BlockSpec((1, F), lambda i: (0, 0)),
        ],
        out_specs=pl.BlockSpec((blk, F), lambda i: (i, 0)),
        out_shape=jax.ShapeDtypeStruct((n, F), jnp.float32),
    )(acc_s, cnt_s, acc_o, cnt_o, x, w1, b1, w2, b2)


def kernel(instance_feats, phrase_feats, connect_mats, phrase_clustered_indexs,
           p_s_W, p_s_b, s_p_W, s_p_b, p_o_W, p_o_b, o_p_W, o_p_b,
           inst_w1_W, inst_w1_b, inst_w2_W, inst_w2_b,
           phra_w1_W, phra_w1_b, phra_w2_W, phra_w2_b):
    inst = instance_feats[0]
    phra = phrase_feats[0]
    conn0 = connect_mats[0, 0]
    conn1 = connect_mats[0, 1]
    clu = phrase_clustered_indexs[0]

    marker = jnp.full((NI * NI,), -1, jnp.int32)
    marker = marker.at[conn0 * NI + conn1].set(jnp.arange(P, dtype=jnp.int32))
    vals = marker[conn0 * NI + conn1]
    keep = vals == jnp.arange(P, dtype=vals.dtype)
    keepf = keep.astype(jnp.float32)

    stp, sti0, sti1 = _stage(phra, inst, clu, conn0, conn1)

    wp = jnp.concatenate(
        [p_s_W[:F], p_o_W[:F], o_p_W[F:], s_p_W[F:]], axis=1
    ).astype(jnp.bfloat16)
    wi0 = jnp.concatenate([p_s_W[F:], s_p_W[:F]], axis=1).astype(jnp.bfloat16)
    wi1 = jnp.concatenate([p_o_W[F:], o_p_W[:F]], axis=1).astype(jnp.bfloat16)
    bcat = jnp.stack([p_s_b, p_o_b, o_p_b, s_p_b], axis=0)
    m1, m2, m3, m4 = _gates(stp, sti0, sti1, wp, wi0, wi1, bcat, keepf[:, None])

    acc_s = jax.ops.segment_sum(m1, conn0, num_segments=NI)
    acc_o = jax.ops.segment_sum(m2, conn1, num_segments=NI)
    cnt_s = jax.ops.segment_sum(keepf, conn0, num_segments=NI)
    cnt_o = jax.ops.segment_sum(keepf, conn1, num_segments=NI)
    acc_o2 = jax.ops.segment_sum(m3, clu, num_segments=NP)
    acc_s2 = jax.ops.segment_sum(m4, clu, num_segments=NP)
    cnt2 = jax.ops.segment_sum(jnp.ones((P,), jnp.float32), clu, num_segments=NP)

    inst_out = _combine(acc_s, cnt_s[:, None], acc_o, cnt_o[:, None], inst,
                        inst_w1_W, inst_w1_b[None], inst_w2_W, inst_w2_b[None])
    phra_out = _combine(acc_o2, cnt2[:, None], acc_s2, cnt2[:, None], phra,
                        phra_w1_W, phra_w1_b[None], phra_w2_W, phra_w2_b[None])
    return inst_out[None], phra_out[None]
